# baseline (speedup 1.0000x reference)
import functools

import jax
import jax.numpy as jnp
import numpy as np
from jax import lax
from jax.experimental import pallas as pl
from jax.experimental.pallas import tpu as pltpu
from jax.experimental.pallas import tpu_sc as plsc

HIDDEN = 64
SEQ = 50
BT = 128
NC, NS = 2, 16
NW = NC * NS
NBUF = 2


def _pos_enc(seq_len, ch):
    channels = int(np.ceil(ch / 2) * 2)
    inv_freq = 1.0 / (10000 ** (np.arange(0, channels, 2).astype(np.float32) / channels))
    pos = np.arange(seq_len).astype(np.float32)
    sin_inp = np.einsum("i,j->ij", pos, inv_freq)
    emb = np.stack((np.sin(sin_inp), np.cos(sin_inp)), axis=-1).reshape(seq_len, channels)
    return emb[:, :ch].astype(np.float32)


def kernel(x, dummy_sigma, embedding):
    del dummy_sigma
    batch, seq = x.shape
    nb = batch // BT
    tasks_per_w = (seq * nb) // NW
    xt = x.T
    pe = _pos_enc(SEQ, HIDDEN)
    pe_sp = jnp.asarray(np.repeat(pe[:, :, None], 16, axis=2))

    mesh = plsc.VectorSubcoreMesh(core_axis_name="c", subcore_axis_name="s")

    @functools.partial(
        pl.kernel,
        out_type=jax.ShapeDtypeStruct((SEQ, HIDDEN // 8, nb, 8, BT),
                                      jnp.float32),
        mesh=mesh,
        compiler_params=pltpu.CompilerParams(use_tc_tiling_on_sc=False, needs_layout_passes=False),
        scratch_types=[
            pltpu.VMEM((NBUF, BT), jnp.int32),
            pltpu.VMEM((SEQ, HIDDEN, 16), jnp.float32),
            pltpu.VMEM((NBUF, BT, HIDDEN), jnp.float32),
            pltpu.VMEM((NBUF, HIDDEN // 8, 8, BT), jnp.float32),
            pltpu.SemaphoreType.DMA((NBUF,)),
            pltpu.SemaphoreType.DMA((NBUF,)),
        ],
    )
    def sc_kernel(table_hbm, xt_hbm, pe_hbm, out_hbm,
                  idx_v, pe_v, ibuf, obuf, gsem, ssem):
        wid = lax.axis_index("s") * NC + lax.axis_index("c")
        base = wid * tasks_per_w
        pltpu.sync_copy(pe_hbm, pe_v)
        rows_q = [jax.lax.iota(jnp.int32, 16) + q * 16 for q in range(8)]

        def task_sb(t):
            return t // nb, lax.rem(t, nb)

        def start_gathers(t, b):
            s, tb = task_sb(t)
            pltpu.sync_copy(xt_hbm.at[s, pl.ds(tb * BT, BT)], idx_v.at[b])
            for k in range(BT // 16):
                iv = idx_v[b, pl.ds(k * 16, 16)]
                pltpu.async_copy(
                    table_hbm.at[iv], ibuf.at[b, pl.ds(k * 16, 16)],
                    gsem.at[b])

        def wait_gathers(b):
            for k in range(BT // 16):
                iv = idx_v[b, pl.ds(k * 16, 16)]
                pltpu.make_async_copy(
                    table_hbm.at[iv], ibuf.at[b, pl.ds(k * 16, 16)],
                    gsem.at[b]).wait()

        def store(t, b):
            s, tb = task_sb(t)
            return pltpu.make_async_copy(
                obuf.at[b], out_hbm.at[s, :, tb], ssem.at[b])

        def transpose_add(t, b):
            s, _ = task_sb(t)
            for h in range(HIDDEN):
                pe_vec = pe_v[s, h]
                cols = jnp.full((16,), h, jnp.int32)
                for q in range(8):
                    v = plsc.load_gather(ibuf.at[b], [rows_q[q], cols])
                    obuf[b, h // 8, h % 8, pl.ds(q * 16, 16)] = v + pe_vec

        for b in range(NBUF):
            start_gathers(base + b, b)

        def main_step(g, carry):
            for b in range(NBUF):
                i = g * NBUF + b
                t = base + i
                wait_gathers(b)

                @pl.when(i >= NBUF)
                def _():
                    store(t - NBUF, b).wait()

                transpose_add(t, b)
                store(t, b).start()
                start_gathers(t + NBUF, b)
            return carry

        lax.fori_loop(0, (tasks_per_w - NBUF) // NBUF, main_step, 0)

        for b in range(NBUF):
            t = base + tasks_per_w - NBUF + b
            wait_gathers(b)
            store(t - NBUF, b).wait()
            transpose_add(t, b)
            store(t, b).start()
        for b in range(NBUF):
            store(base + tasks_per_w - NBUF + b, b).wait()

    out5 = sc_kernel(embedding, xt, pe_sp)
    return out5.transpose(2, 4, 0, 1, 3).reshape(batch, seq, HIDDEN)


# speedup vs baseline: 1.3281x; 1.3281x over previous
import functools

import jax
import jax.numpy as jnp
import numpy as np
from jax import lax
from jax.experimental import pallas as pl
from jax.experimental.pallas import tpu as pltpu
from jax.experimental.pallas import tpu_sc as plsc

HIDDEN = 64
SEQ = 50
UNIT = 8 * SEQ
NC, NS = 2, 16
NW = NC * NS
NBUF = 2


def _pos_enc(seq_len: int, ch: int) -> np.ndarray:
    channels = int(np.ceil(ch / 2) * 2)
    inv_freq = 1.0 / (10000 ** (np.arange(0, channels, 2).astype(np.float32) / channels))
    pos = np.arange(seq_len).astype(np.float32)
    sin_inp = np.einsum("i,j->ij", pos, inv_freq)
    emb = np.stack((np.sin(sin_inp), np.cos(sin_inp)), axis=-1).reshape(seq_len, channels)
    return emb[:, :ch].astype(np.float32)


def kernel(x, dummy_sigma, embedding):
    del dummy_sigma
    n_units = (x.shape[0] * x.shape[1]) // UNIT
    units_per_w = n_units // NW
    idx2d = x.reshape(n_units, UNIT)
    pe2 = jnp.asarray(_pos_enc(SEQ, HIDDEN))

    mesh = plsc.VectorSubcoreMesh(core_axis_name="c", subcore_axis_name="s")

    @functools.partial(
        pl.kernel,
        out_type=jax.ShapeDtypeStruct((n_units * UNIT, HIDDEN), jnp.float32),
        mesh=mesh,
        compiler_params=pltpu.CompilerParams(use_tc_tiling_on_sc=False),
        scratch_types=[
            pltpu.VMEM((units_per_w, UNIT), jnp.int32),
            pltpu.VMEM((SEQ, HIDDEN), jnp.float32),
            pltpu.VMEM((NBUF, UNIT, HIDDEN), jnp.float32),
            pltpu.VMEM((NBUF, UNIT, HIDDEN), jnp.float32),
            pltpu.SemaphoreType.DMA((NBUF, 4)),
            pltpu.SemaphoreType.DMA((NBUF,)),
        ],
    )
    def sc_kernel(table_hbm, idx_hbm, pe_hbm, out_hbm,
                  idx_v, pe_v, buf, obuf, gsem, ssem):
        wid = lax.axis_index("s") * NC + lax.axis_index("c")
        base = wid * units_per_w
        pltpu.sync_copy(idx_hbm.at[pl.ds(base, units_per_w)], idx_v)
        pltpu.sync_copy(pe_hbm, pe_v)

        def start_gathers(u, b):
            for k in range(UNIT // 16):
                iv = idx_v[u, pl.ds(k * 16, 16)]
                pltpu.async_copy(
                    table_hbm.at[iv], buf.at[b, pl.ds(k * 16, 16)],
                    gsem.at[b, k % 4])

        def wait_gathers(u, b):
            for k in range(UNIT // 16):
                pltpu.make_async_copy(
                    table_hbm.at[idx_v[u, pl.ds(k * 16, 16)]],
                    buf.at[b, pl.ds(k * 16, 16)],
                    gsem.at[b, k % 4]).wait()

        def store(u, b):
            return pltpu.make_async_copy(
                obuf.at[b], out_hbm.at[pl.ds((base + u) * UNIT, UNIT)],
                ssem.at[b])

        def add_pe(b):
            def add_row(i, c):
                for r in range(UNIT // SEQ):
                    for g in range(HIDDEN // 16):
                        sl = pl.ds(g * 16, 16)
                        obuf[b, r * SEQ + i, sl] = (
                            buf[b, r * SEQ + i, sl] + pe_v[i, sl])
                return c
            lax.fori_loop(0, SEQ, add_row, 0)

        for b in range(NBUF):
            start_gathers(b, b)

        def main_step(g, carry):
            for b in range(NBUF):
                j = g * NBUF + b
                wait_gathers(j, b)

                @pl.when(j >= NBUF)
                def _():
                    store(j - NBUF, b).wait()

                add_pe(b)
                store(j, b).start()
                start_gathers(j + NBUF, b)
            return carry

        lax.fori_loop(0, (units_per_w - NBUF) // NBUF, main_step, 0)

        for b in range(NBUF):
            j = units_per_w - NBUF + b
            wait_gathers(j, b)
            store(j - NBUF, b).wait()
            add_pe(b)
            store(j, b).start()
        for b in range(NBUF):
            store(units_per_w - NBUF + b, b).wait()

    out = sc_kernel(embedding, idx2d, pe2)
    return out.reshape(x.shape[0], x.shape[1], HIDDEN)
